# Initial kernel scaffold; baseline (speedup 1.0000x reference)
import jax
import jax.numpy as jnp
from jax.experimental import pallas as pl


def kernel(sp_x, sp_edge_index, sp_batch, kp_x, kp_edge_index, kp_batch, params):
    raise NotImplementedError("write your pallas kernel here")


# speedup vs baseline: 1.0065x; 1.0065x over previous
import functools

import jax
import jax.numpy as jnp
from jax.experimental import pallas as pl
from jax.experimental.pallas import tpu as pltpu

N = 50000
G = 64
H = 64
C = 100


def _ln(x, g, b):
    mu = jnp.mean(x, axis=-1, keepdims=True)
    var = jnp.var(x, axis=-1, keepdims=True)
    return (x - mu) / jnp.sqrt(var + 1e-5) * g + b


def _fusion_body(sp_ref, kp_ref, w1_ref, b1_ref, g1_ref, be1_ref, w2_ref,
                 b2_ref, cw1_ref, cb1_ref, cg_ref, cbe_ref, cw2_ref, cb2_ref,
                 out_ref):
    out_sp = sp_ref[...]
    out_kp = kp_ref[...]

    def l2n(v):
        nrm = jnp.sqrt(jnp.sum(v * v, axis=1, keepdims=True))
        return v / jnp.maximum(nrm, 1e-12)

    out_sp = l2n(out_sp)
    out_kp = l2n(out_kp)
    gin = jnp.concatenate([out_sp, out_kp], axis=-1)
    g1 = gin @ w1_ref[...] + b1_ref[...]
    g1 = jax.nn.relu(_ln(g1, g1_ref[...], be1_ref[...]))
    gate = jax.nn.sigmoid(g1 @ w2_ref[...] + b2_ref[...])
    fused = gate * out_sp + (1.0 - gate) * out_kp
    h1 = fused @ cw1_ref[...] + cb1_ref[...]
    h1 = jax.nn.relu(_ln(h1, cg_ref[...], cbe_ref[...]))
    out_ref[...] = h1 @ cw2_ref[...] + cb2_ref[...]


def _fusion(out_sp, out_kp, gp, cp):
    args = [out_sp, out_kp, gp['W1'], gp['b1'].reshape(1, H), gp['g'].reshape(1, H),
            gp['be'].reshape(1, H), gp['W2'], gp['b2'].reshape(1, H),
            cp['W1'], cp['b1'].reshape(1, H), cp['g'].reshape(1, H),
            cp['be'].reshape(1, H), cp['W2'], cp['b2'].reshape(1, C)]
    return pl.pallas_call(
        _fusion_body,
        out_shape=jax.ShapeDtypeStruct((G, C), jnp.float32),
    )(*args)


def _seg_softmax(e, seg, n):
    m = jax.ops.segment_max(e, seg, num_segments=n)
    m = jnp.where(jnp.isfinite(m), m, 0.0)
    ex = jnp.exp(e - m[seg])
    s = jax.ops.segment_sum(ex, seg, num_segments=n)
    return ex / (s[seg] + 1e-16)


def _mean_pool(x, seg, n):
    s = jax.ops.segment_sum(x, seg, num_segments=n)
    cnt = jax.ops.segment_sum(jnp.ones((x.shape[0],), jnp.float32), seg,
                              num_segments=n)
    return s / jnp.maximum(cnt, 1.0)[:, None]


def kernel(sp_x, sp_edge_index, sp_batch, kp_x, kp_edge_index, kp_batch,
           params):
    x = sp_x
    src, dst = sp_edge_index[0], sp_edge_index[1]
    ones_e = jnp.ones((src.shape[0],), jnp.float32)
    for lp in params['sp']:
        s = jax.ops.segment_sum(x[src], dst, num_segments=N)
        cnt = jax.ops.segment_sum(ones_e, dst, num_segments=N)
        mean = s / jnp.maximum(cnt, 1.0)[:, None]
        x = mean @ lp['Wl'] + x @ lp['Wr'] + lp['b']
        x = jax.nn.relu(_ln(x, lp['g'], lp['be']))
    out_sp = _mean_pool(x, sp_batch, G)

    x = kp_x
    ksrc, kdst = kp_edge_index[0], kp_edge_index[1]
    kp_cfg = [(4, 16, True), (4, 16, True), (1, H, False)]
    for lp, (h, oh, cc) in zip(params['kp'], kp_cfg):
        hx = (x @ lp['W']).reshape(N, h, oh)
        a_s = jnp.sum(hx * lp['asrc'][None], axis=-1)
        a_d = jnp.sum(hx * lp['adst'][None], axis=-1)
        e = jax.nn.leaky_relu(a_s[ksrc] + a_d[kdst], 0.2)
        alpha = _seg_softmax(e, kdst, N)
        out = jax.ops.segment_sum(hx[ksrc] * alpha[:, :, None], kdst,
                                  num_segments=N)
        x = out.reshape(N, h * oh) if cc else out.mean(axis=1)
        x = x + lp['b']
        x = jax.nn.relu(_ln(x, lp['g'], lp['be']))
    out_kp = _mean_pool(x, kp_batch, G)

    return _fusion(out_sp, out_kp, params['gate'], params['cls'])


# speedup vs baseline: 22.9234x; 22.7760x over previous
import functools

import jax
import jax.numpy as jnp
from jax import lax
from jax.experimental import pallas as pl
from jax.experimental.pallas import tpu as pltpu
from jax.experimental.pallas import tpu_sc as plsc

N = 50000
E = 800000
G = 64
H = 64
C = 100

_CH = 128
_NCH = E // _CH
_NS = 16


def _make_seg_sum(d, col_split):
    mesh = plsc.VectorSubcoreMesh(core_axis_name="c", subcore_axis_name="s")

    @functools.partial(
        pl.kernel,
        out_type=jax.ShapeDtypeStruct((2, N, d), jnp.float32),
        mesh=mesh,
        scratch_types=[
            pltpu.VMEM((2, _CH), jnp.int32),
            pltpu.VMEM((_CH, d), jnp.float32),
            pltpu.VMEM_SHARED((N, d), jnp.float32),
            pltpu.SemaphoreType.DMA,
        ],
        compiler_params=pltpu.CompilerParams(use_tc_tiling_on_sc=False),
    )
    def k(tab, idx, zeros, out, idx_v, rows_v, acc, sem):
        c = lax.axis_index("c")
        s = lax.axis_index("s")

        @pl.when(s == 0)
        def _zero():
            pltpu.sync_copy(zeros, acc)

        plsc.subcore_barrier()

        tab_r = tab.at[c] if col_split else tab.at[0]
        if col_split:
            start = s
            step = _NS
        else:
            start = c * _NS + s
            step = 2 * _NS
        n_my = (_NCH - start + step - 1) // step

        def body(i, carry):
            j = start + i * step
            pltpu.sync_copy(idx.at[j], idx_v)
            pltpu.async_copy(tab_r.at[idx_v.at[0]], rows_v, sem).wait()
            pltpu.sync_copy(rows_v, acc.at[idx_v.at[1]], add=True)
            return carry

        lax.fori_loop(0, n_my, body, 0)
        plsc.subcore_barrier()

        @pl.when(s == 0)
        def _write():
            pltpu.sync_copy(acc, out.at[c])

    return k


_seg_sum8 = _make_seg_sum(8, col_split=False)
_seg_sum32 = _make_seg_sum(32, col_split=True)


def _make_gat_edge():
    mesh = plsc.VectorSubcoreMesh(core_axis_name="c", subcore_axis_name="s")

    @functools.partial(
        pl.kernel,
        out_type=(jax.ShapeDtypeStruct((_NCH, _CH, 16), jnp.float32),
                  jax.ShapeDtypeStruct((2, N, 16), jnp.float32)),
        mesh=mesh,
        scratch_types=[
            pltpu.VMEM((2, _CH), jnp.int32),
            pltpu.VMEM((_CH, 16), jnp.float32),
            pltpu.VMEM((_CH, 16), jnp.float32),
            pltpu.VMEM((16,), jnp.float32),
            pltpu.VMEM_SHARED((N, 16), jnp.float32),
            pltpu.SemaphoreType.DMA,
        ],
        compiler_params=pltpu.CompilerParams(use_tc_tiling_on_sc=False),
    )
    def k(asrc, adst, mvec, idx, zeros, w_out, s_out, idx_v, a_v, b_v, m_v,
          acc, sem):
        c = lax.axis_index("c")
        s = lax.axis_index("s")

        @pl.when(s == 0)
        def _zero():
            pltpu.sync_copy(zeros, acc)

        pltpu.sync_copy(mvec, m_v)
        plsc.subcore_barrier()

        start = c * _NS + s
        step = 2 * _NS
        n_my = (_NCH - start + step - 1) // step

        def body(i, carry):
            j = start + i * step
            pltpu.sync_copy(idx.at[j], idx_v)
            pltpu.async_copy(asrc.at[idx_v.at[0]], a_v, sem).wait()
            pltpu.async_copy(adst.at[idx_v.at[1]], b_v, sem).wait()
            mv = m_v[...]

            def ebody(i2, c2):
                e = a_v[i2] + b_v[i2]
                e = jnp.maximum(e, 0.2 * e)
                a_v[i2] = jnp.exp(e - mv)
                return c2

            lax.fori_loop(0, _CH, ebody, 0)
            pltpu.sync_copy(a_v, acc.at[idx_v.at[1]], add=True)
            pltpu.sync_copy(a_v, w_out.at[j])
            return carry

        lax.fori_loop(0, n_my, body, 0)
        plsc.subcore_barrier()

        @pl.when(s == 0)
        def _write():
            pltpu.sync_copy(acc, s_out.at[c])

    return k


def _make_wseg(two_heads):
    mesh = plsc.VectorSubcoreMesh(core_axis_name="c", subcore_axis_name="s")

    @functools.partial(
        pl.kernel,
        out_type=jax.ShapeDtypeStruct((2, N, 32), jnp.float32),
        mesh=mesh,
        scratch_types=[
            pltpu.VMEM((2, _CH), jnp.int32),
            pltpu.VMEM((_CH, 32), jnp.float32),
            pltpu.VMEM((_CH, 16), jnp.float32),
            pltpu.VMEM_SHARED((N, 32), jnp.float32),
            pltpu.SemaphoreType.DMA,
        ],
        compiler_params=pltpu.CompilerParams(use_tc_tiling_on_sc=False),
    )
    def k(tab, w_in, idx, zeros, out, idx_v, rows_v, w_v, acc, sem):
        c = lax.axis_index("c")
        s = lax.axis_index("s")

        @pl.when(s == 0)
        def _zero():
            pltpu.sync_copy(zeros, acc)

        plsc.subcore_barrier()

        def body(i, carry):
            j = s + i * _NS
            pltpu.sync_copy(idx.at[j], idx_v)
            pltpu.async_copy(tab.at[c].at[idx_v.at[0]], rows_v, sem).wait()
            pltpu.sync_copy(w_in.at[j], w_v)

            def ebody(i2, c2):
                wrow = w_v[i2]
                if two_heads:
                    w0 = jnp.where(c == 0, wrow[0], wrow[2])
                    w1 = jnp.where(c == 0, wrow[1], wrow[3])
                else:
                    w0 = wrow[0]
                    w1 = w0
                rows_v[i2, pl.ds(0, 16)] = rows_v[i2, pl.ds(0, 16)] * w0
                rows_v[i2, pl.ds(16, 16)] = rows_v[i2, pl.ds(16, 16)] * w1
                return c2

            lax.fori_loop(0, _CH, ebody, 0)
            pltpu.sync_copy(rows_v, acc.at[idx_v.at[1]], add=True)
            return carry

        n_my = (_NCH - s + _NS - 1) // _NS
        lax.fori_loop(0, n_my, body, 0)
        plsc.subcore_barrier()

        @pl.when(s == 0)
        def _write():
            pltpu.sync_copy(acc, out.at[c])

    return k


_gat_edge = _make_gat_edge()
_wseg2 = _make_wseg(True)
_wseg1 = _make_wseg(False)


def _edge_chunks(ei):
    ei = ei.astype(jnp.int32)
    return jnp.transpose(ei.reshape(2, _NCH, _CH), (1, 0, 2))


def _ln(x, g, b):
    mu = jnp.mean(x, axis=-1, keepdims=True)
    var = jnp.var(x, axis=-1, keepdims=True)
    return (x - mu) / jnp.sqrt(var + 1e-5) * g + b


def _fusion_body(sp_ref, kp_ref, w1_ref, b1_ref, g1_ref, be1_ref, w2_ref,
                 b2_ref, cw1_ref, cb1_ref, cg_ref, cbe_ref, cw2_ref, cb2_ref,
                 out_ref):
    out_sp = sp_ref[...]
    out_kp = kp_ref[...]

    def l2n(v):
        nrm = jnp.sqrt(jnp.sum(v * v, axis=1, keepdims=True))
        return v / jnp.maximum(nrm, 1e-12)

    out_sp = l2n(out_sp)
    out_kp = l2n(out_kp)
    gin = jnp.concatenate([out_sp, out_kp], axis=-1)
    g1 = gin @ w1_ref[...] + b1_ref[...]
    g1 = jax.nn.relu(_ln(g1, g1_ref[...], be1_ref[...]))
    gate = jax.nn.sigmoid(g1 @ w2_ref[...] + b2_ref[...])
    fused = gate * out_sp + (1.0 - gate) * out_kp
    h1 = fused @ cw1_ref[...] + cb1_ref[...]
    h1 = jax.nn.relu(_ln(h1, cg_ref[...], cbe_ref[...]))
    out_ref[...] = h1 @ cw2_ref[...] + cb2_ref[...]


def _fusion(out_sp, out_kp, gp, cp):
    args = [out_sp, out_kp, gp['W1'], gp['b1'].reshape(1, H), gp['g'].reshape(1, H),
            gp['be'].reshape(1, H), gp['W2'], gp['b2'].reshape(1, H),
            cp['W1'], cp['b1'].reshape(1, H), cp['g'].reshape(1, H),
            cp['be'].reshape(1, H), cp['W2'], cp['b2'].reshape(1, C)]
    return pl.pallas_call(
        _fusion_body,
        out_shape=jax.ShapeDtypeStruct((G, C), jnp.float32),
    )(*args)


def _seg_softmax(e, seg, n):
    m = jax.ops.segment_max(e, seg, num_segments=n)
    m = jnp.where(jnp.isfinite(m), m, 0.0)
    ex = jnp.exp(e - m[seg])
    s = jax.ops.segment_sum(ex, seg, num_segments=n)
    return ex / (s[seg] + 1e-16)


def _mean_pool(x, seg, n):
    s = jax.ops.segment_sum(x, seg, num_segments=n)
    cnt = jax.ops.segment_sum(jnp.ones((x.shape[0],), jnp.float32), seg,
                              num_segments=n)
    return s / jnp.maximum(cnt, 1.0)[:, None]


def kernel(sp_x, sp_edge_index, sp_batch, kp_x, kp_edge_index, kp_batch,
           params):
    sp_chunks = _edge_chunks(sp_edge_index)
    x = sp_x
    zeros8 = jnp.zeros((N, 8), jnp.float32)
    zeros32 = jnp.zeros((N, 32), jnp.float32)
    for li, lp in enumerate(params['sp']):
        if li == 0:
            tab = jnp.concatenate(
                [x, jnp.zeros((N, 1), jnp.float32),
                 jnp.ones((N, 1), jnp.float32)], axis=1)
            part = _seg_sum8(tab[None], sp_chunks, zeros8)
            agg = part[0] + part[1]
            s = agg[:, :6]
            cnt = agg[:, 7]
        else:
            tab = jnp.transpose(x.reshape(N, 2, 32), (1, 0, 2))
            halves = _seg_sum32(tab, sp_chunks, zeros32)
            s = jnp.concatenate([halves[0], halves[1]], axis=1)
        mean = s / jnp.maximum(cnt, 1.0)[:, None]
        x = mean @ lp['Wl'] + x @ lp['Wr'] + lp['b']
        x = jax.nn.relu(_ln(x, lp['g'], lp['be']))
    out_sp = _mean_pool(x, sp_batch, G)

    kp_chunks = _edge_chunks(kp_edge_index)
    x = kp_x
    zeros16 = jnp.zeros((N, 16), jnp.float32)
    kp_cfg = [(4, 16, True), (4, 16, True), (1, H, False)]
    for lp, (h, oh, cc) in zip(params['kp'], kp_cfg):
        hx = x @ lp['W']
        hxr = hx.reshape(N, h, oh)
        a_s = jnp.sum(hxr * lp['asrc'][None], axis=-1)
        a_d = jnp.sum(hxr * lp['adst'][None], axis=-1)
        m_h = jax.nn.leaky_relu(
            jnp.max(a_s, axis=0) + jnp.max(a_d, axis=0), 0.2)
        mvec = jnp.zeros((16,), jnp.float32).at[:h].set(m_h)
        pad = jnp.zeros((N, 16 - h), jnp.float32)
        asrc_tab = jnp.concatenate([a_s, pad], axis=1)
        adst_tab = jnp.concatenate([a_d, pad], axis=1)
        w_chunks, s_part = _gat_edge(asrc_tab, adst_tab, mvec, kp_chunks,
                                     zeros16)
        denom = (s_part[0] + s_part[1])[:, :h]
        tab = jnp.transpose(hx.reshape(N, 2, 32), (1, 0, 2))
        wseg = _wseg2 if h == 4 else _wseg1
        halves = wseg(tab, w_chunks, kp_chunks, zeros32)
        acc = jnp.concatenate([halves[0], halves[1]], axis=1)
        out = acc.reshape(N, h, oh) / (denom[:, :, None] + 1e-16)
        x = out.reshape(N, h * oh) if cc else out.mean(axis=1)
        x = x + lp['b']
        x = jax.nn.relu(_ln(x, lp['g'], lp['be']))
    out_kp = _mean_pool(x, kp_batch, G)

    return _fusion(out_sp, out_kp, params['gate'], params['cls'])


# speedup vs baseline: 23.7081x; 1.0342x over previous
import functools

import jax
import jax.numpy as jnp
from jax import lax
from jax.experimental import pallas as pl
from jax.experimental.pallas import tpu as pltpu
from jax.experimental.pallas import tpu_sc as plsc

N = 50000
E = 800000
G = 64
H = 64
C = 100

_CH = 128
_NCH = E // _CH
_NS = 16


def _make_seg_sum(d, col_split):
    mesh = plsc.VectorSubcoreMesh(core_axis_name="c", subcore_axis_name="s")

    @functools.partial(
        pl.kernel,
        out_type=jax.ShapeDtypeStruct((2, N, d), jnp.float32),
        mesh=mesh,
        scratch_types=[
            pltpu.VMEM((2, _CH), jnp.int32),
            pltpu.VMEM((_CH, d), jnp.float32),
            pltpu.VMEM_SHARED((N, d), jnp.float32),
            pltpu.SemaphoreType.DMA,
        ],
        compiler_params=pltpu.CompilerParams(use_tc_tiling_on_sc=False),
    )
    def k(tab, idx, zeros, out, idx_v, rows_v, acc, sem):
        c = lax.axis_index("c")
        s = lax.axis_index("s")

        @pl.when(s == 0)
        def _zero():
            pltpu.sync_copy(zeros, acc)

        plsc.subcore_barrier()

        tab_r = tab.at[c] if col_split else tab.at[0]
        if col_split:
            start = s
            step = _NS
        else:
            start = c * _NS + s
            step = 2 * _NS
        n_my = (_NCH - start + step - 1) // step

        def body(i, carry):
            j = start + i * step
            pltpu.sync_copy(idx.at[j], idx_v)
            pltpu.async_copy(tab_r.at[idx_v.at[0]], rows_v, sem).wait()
            pltpu.sync_copy(rows_v, acc.at[idx_v.at[1]], add=True)
            return carry

        lax.fori_loop(0, n_my, body, 0)
        plsc.subcore_barrier()

        @pl.when(s == 0)
        def _write():
            pltpu.sync_copy(acc, out.at[c])

    return k


_seg_sum8 = _make_seg_sum(8, col_split=False)
_seg_sum32 = _make_seg_sum(32, col_split=True)


def _make_gat_edge():
    mesh = plsc.VectorSubcoreMesh(core_axis_name="c", subcore_axis_name="s")

    @functools.partial(
        pl.kernel,
        out_type=(jax.ShapeDtypeStruct((_NCH, _CH, 16), jnp.float32),
                  jax.ShapeDtypeStruct((2, N, 16), jnp.float32)),
        mesh=mesh,
        scratch_types=[
            pltpu.VMEM((2, _CH), jnp.int32),
            pltpu.VMEM((_CH, 16), jnp.float32),
            pltpu.VMEM((_CH, 16), jnp.float32),
            pltpu.VMEM((16,), jnp.float32),
            pltpu.VMEM_SHARED((N, 16), jnp.float32),
            pltpu.SemaphoreType.DMA,
        ],
        compiler_params=pltpu.CompilerParams(use_tc_tiling_on_sc=False),
    )
    def k(asrc, adst, mvec, idx, zeros, w_out, s_out, idx_v, a_v, b_v, m_v,
          acc, sem):
        c = lax.axis_index("c")
        s = lax.axis_index("s")

        @pl.when(s == 0)
        def _zero():
            pltpu.sync_copy(zeros, acc)

        pltpu.sync_copy(mvec, m_v)
        plsc.subcore_barrier()

        start = c * _NS + s
        step = 2 * _NS
        n_my = (_NCH - start + step - 1) // step

        def body(i, carry):
            j = start + i * step
            pltpu.sync_copy(idx.at[j], idx_v)
            pltpu.async_copy(asrc.at[idx_v.at[0]], a_v, sem).wait()
            pltpu.async_copy(adst.at[idx_v.at[1]], b_v, sem).wait()
            mv = m_v[...]

            def ebody(i2, c2):
                e = a_v[i2] + b_v[i2]
                e = jnp.maximum(e, 0.2 * e)
                a_v[i2] = jnp.exp(e - mv)
                return c2

            lax.fori_loop(0, _CH, ebody, 0)
            pltpu.sync_copy(a_v, acc.at[idx_v.at[1]], add=True)
            pltpu.sync_copy(a_v, w_out.at[j])
            return carry

        lax.fori_loop(0, n_my, body, 0)
        plsc.subcore_barrier()

        @pl.when(s == 0)
        def _write():
            pltpu.sync_copy(acc, s_out.at[c])

    return k


def _make_wseg(two_heads):
    mesh = plsc.VectorSubcoreMesh(core_axis_name="c", subcore_axis_name="s")

    @functools.partial(
        pl.kernel,
        out_type=jax.ShapeDtypeStruct((2, N, 32), jnp.float32),
        mesh=mesh,
        scratch_types=[
            pltpu.VMEM((2, _CH), jnp.int32),
            pltpu.VMEM((_CH, 32), jnp.float32),
            pltpu.VMEM((_CH, 16), jnp.float32),
            pltpu.VMEM_SHARED((N, 32), jnp.float32),
            pltpu.SemaphoreType.DMA,
        ],
        compiler_params=pltpu.CompilerParams(use_tc_tiling_on_sc=False),
    )
    def k(tab, w_in, idx, zeros, out, idx_v, rows_v, w_v, acc, sem):
        c = lax.axis_index("c")
        s = lax.axis_index("s")

        @pl.when(s == 0)
        def _zero():
            pltpu.sync_copy(zeros, acc)

        plsc.subcore_barrier()

        def body(i, carry):
            j = s + i * _NS
            pltpu.sync_copy(idx.at[j], idx_v)
            pltpu.async_copy(tab.at[c].at[idx_v.at[0]], rows_v, sem).wait()
            pltpu.sync_copy(w_in.at[j], w_v)

            def ebody(i2, c2):
                wrow = w_v[i2]
                if two_heads:
                    w0 = jnp.where(c == 0, wrow[0], wrow[2])
                    w1 = jnp.where(c == 0, wrow[1], wrow[3])
                else:
                    w0 = wrow[0]
                    w1 = w0
                rows_v[i2, pl.ds(0, 16)] = rows_v[i2, pl.ds(0, 16)] * w0
                rows_v[i2, pl.ds(16, 16)] = rows_v[i2, pl.ds(16, 16)] * w1
                return c2

            lax.fori_loop(0, _CH, ebody, 0)
            pltpu.sync_copy(rows_v, acc.at[idx_v.at[1]], add=True)
            return carry

        n_my = (_NCH - s + _NS - 1) // _NS
        lax.fori_loop(0, n_my, body, 0)
        plsc.subcore_barrier()

        @pl.when(s == 0)
        def _write():
            pltpu.sync_copy(acc, out.at[c])

    return k


_gat_edge = _make_gat_edge()
_wseg2 = _make_wseg(True)
_wseg1 = _make_wseg(False)


def _edge_chunks(ei):
    ei = ei.astype(jnp.int32)
    return jnp.transpose(ei.reshape(2, _NCH, _CH), (1, 0, 2))


def _ln(x, g, b):
    mu = jnp.mean(x, axis=-1, keepdims=True)
    var = jnp.var(x, axis=-1, keepdims=True)
    return (x - mu) / jnp.sqrt(var + 1e-5) * g + b


_PB = 1000
_NPB = N // _PB


def _pool_fusion_body(xsp_ref, xkp_ref, spb_ref, kpb_ref, w1_ref, b1_ref,
                      g1_ref, be1_ref, w2_ref, b2_ref, cw1_ref, cb1_ref,
                      cg_ref, cbe_ref, cw2_ref, cb2_ref, out_ref,
                      accs, acck, cnts, cntk):
    i = pl.program_id(0)

    @pl.when(i == 0)
    def _zero():
        accs[...] = jnp.zeros_like(accs)
        acck[...] = jnp.zeros_like(acck)
        cnts[...] = jnp.zeros_like(cnts)
        cntk[...] = jnp.zeros_like(cntk)

    gids = lax.broadcasted_iota(jnp.int32, (G, _PB), 0)
    dims = (((1,), (0,)), ((), ()))
    ohs = (spb_ref[0] == gids).astype(jnp.float32)
    accs[...] += lax.dot_general(ohs, xsp_ref[...], dims,
                                 preferred_element_type=jnp.float32)
    cnts[...] += jnp.sum(ohs, axis=1, keepdims=True)
    ohk = (kpb_ref[0] == gids).astype(jnp.float32)
    acck[...] += lax.dot_general(ohk, xkp_ref[...], dims,
                                 preferred_element_type=jnp.float32)
    cntk[...] += jnp.sum(ohk, axis=1, keepdims=True)

    @pl.when(i == _NPB - 1)
    def _finish():
        out_sp = accs[...] / jnp.maximum(cnts[...], 1.0)
        out_kp = acck[...] / jnp.maximum(cntk[...], 1.0)

        def l2n(v):
            nrm = jnp.sqrt(jnp.sum(v * v, axis=1, keepdims=True))
            return v / jnp.maximum(nrm, 1e-12)

        out_sp = l2n(out_sp)
        out_kp = l2n(out_kp)
        gin = jnp.concatenate([out_sp, out_kp], axis=-1)
        g1 = gin @ w1_ref[...] + b1_ref[...]
        g1 = jax.nn.relu(_ln(g1, g1_ref[...], be1_ref[...]))
        gate = jax.nn.sigmoid(g1 @ w2_ref[...] + b2_ref[...])
        fused = gate * out_sp + (1.0 - gate) * out_kp
        h1 = fused @ cw1_ref[...] + cb1_ref[...]
        h1 = jax.nn.relu(_ln(h1, cg_ref[...], cbe_ref[...]))
        out_ref[...] = h1 @ cw2_ref[...] + cb2_ref[...]


def _pool_fusion(x_sp, sp_batch, x_kp, kp_batch, gp, cp):
    spb = sp_batch.astype(jnp.int32).reshape(_NPB, 1, _PB)
    kpb = kp_batch.astype(jnp.int32).reshape(_NPB, 1, _PB)
    args = [x_sp, x_kp, spb, kpb,
            gp['W1'], gp['b1'].reshape(1, H), gp['g'].reshape(1, H),
            gp['be'].reshape(1, H), gp['W2'], gp['b2'].reshape(1, H),
            cp['W1'], cp['b1'].reshape(1, H), cp['g'].reshape(1, H),
            cp['be'].reshape(1, H), cp['W2'], cp['b2'].reshape(1, C)]
    node_spec = pl.BlockSpec((_PB, H), lambda i: (i, 0))
    batch_spec = pl.BlockSpec((1, 1, _PB), lambda i: (i, 0, 0))

    def full(a):
        nd = a.ndim
        return pl.BlockSpec(a.shape, lambda i, _n=nd: (0,) * _n)

    in_specs = [node_spec, node_spec, batch_spec, batch_spec] + [
        full(a) for a in args[4:]]
    return pl.pallas_call(
        _pool_fusion_body,
        grid=(_NPB,),
        in_specs=in_specs,
        out_specs=pl.BlockSpec((G, C), lambda i: (0, 0)),
        out_shape=jax.ShapeDtypeStruct((G, C), jnp.float32),
        scratch_shapes=[
            pltpu.VMEM((G, H), jnp.float32),
            pltpu.VMEM((G, H), jnp.float32),
            pltpu.VMEM((G, 1), jnp.float32),
            pltpu.VMEM((G, 1), jnp.float32),
        ],
    )(*args)


def kernel(sp_x, sp_edge_index, sp_batch, kp_x, kp_edge_index, kp_batch,
           params):
    sp_chunks = _edge_chunks(sp_edge_index)
    x = sp_x
    zeros8 = jnp.zeros((N, 8), jnp.float32)
    zeros32 = jnp.zeros((N, 32), jnp.float32)
    for li, lp in enumerate(params['sp']):
        if li == 0:
            tab = jnp.concatenate(
                [x, jnp.zeros((N, 1), jnp.float32),
                 jnp.ones((N, 1), jnp.float32)], axis=1)
            part = _seg_sum8(tab[None], sp_chunks, zeros8)
            agg = part[0] + part[1]
            s = agg[:, :6]
            cnt = agg[:, 7]
        else:
            tab = jnp.transpose(x.reshape(N, 2, 32), (1, 0, 2))
            halves = _seg_sum32(tab, sp_chunks, zeros32)
            s = jnp.concatenate([halves[0], halves[1]], axis=1)
        mean = s / jnp.maximum(cnt, 1.0)[:, None]
        x = mean @ lp['Wl'] + x @ lp['Wr'] + lp['b']
        x = jax.nn.relu(_ln(x, lp['g'], lp['be']))
    x_sp = x

    kp_chunks = _edge_chunks(kp_edge_index)
    x = kp_x
    zeros16 = jnp.zeros((N, 16), jnp.float32)
    kp_cfg = [(4, 16, True), (4, 16, True), (1, H, False)]
    for lp, (h, oh, cc) in zip(params['kp'], kp_cfg):
        hx = x @ lp['W']
        hxr = hx.reshape(N, h, oh)
        a_s = jnp.sum(hxr * lp['asrc'][None], axis=-1)
        a_d = jnp.sum(hxr * lp['adst'][None], axis=-1)
        m_h = jax.nn.leaky_relu(
            jnp.max(a_s, axis=0) + jnp.max(a_d, axis=0), 0.2)
        mvec = jnp.zeros((16,), jnp.float32).at[:h].set(m_h)
        pad = jnp.zeros((N, 16 - h), jnp.float32)
        asrc_tab = jnp.concatenate([a_s, pad], axis=1)
        adst_tab = jnp.concatenate([a_d, pad], axis=1)
        w_chunks, s_part = _gat_edge(asrc_tab, adst_tab, mvec, kp_chunks,
                                     zeros16)
        denom = (s_part[0] + s_part[1])[:, :h]
        tab = jnp.transpose(hx.reshape(N, 2, 32), (1, 0, 2))
        wseg = _wseg2 if h == 4 else _wseg1
        halves = wseg(tab, w_chunks, kp_chunks, zeros32)
        acc = jnp.concatenate([halves[0], halves[1]], axis=1)
        out = acc.reshape(N, h, oh) / (denom[:, :, None] + 1e-16)
        x = out.reshape(N, h * oh) if cc else out.mean(axis=1)
        x = x + lp['b']
        x = jax.nn.relu(_ln(x, lp['g'], lp['be']))

    return _pool_fusion(x_sp, sp_batch, x, kp_batch, params['gate'],
                        params['cls'])


# speedup vs baseline: 25.9100x; 1.0929x over previous
import functools

import jax
import jax.numpy as jnp
from jax import lax
from jax.experimental import pallas as pl
from jax.experimental.pallas import tpu as pltpu
from jax.experimental.pallas import tpu_sc as plsc

N = 50000
E = 800000
G = 64
H = 64
C = 100

_CH = 128
_NCH = E // _CH
_NS = 16


_GRP = 5
_NSUP = _NCH // _GRP


def _make_seg_sum(d, col_split):
    mesh = plsc.VectorSubcoreMesh(core_axis_name="c", subcore_axis_name="s")

    @functools.partial(
        pl.kernel,
        out_type=jax.ShapeDtypeStruct((2, N, d), jnp.float32),
        mesh=mesh,
        scratch_types=[
            pltpu.VMEM((2, _GRP, _CH), jnp.int32),
            pltpu.VMEM((_GRP, _CH, d), jnp.float32),
            pltpu.VMEM_SHARED((N, d), jnp.float32),
            pltpu.SemaphoreType.DMA,
        ],
        compiler_params=pltpu.CompilerParams(use_tc_tiling_on_sc=False),
    )
    def k(tab, idx, zeros, out, idx_v, rows_v, acc, sem):
        c = lax.axis_index("c")
        s = lax.axis_index("s")

        @pl.when(s == 0)
        def _zero():
            pltpu.sync_copy(zeros, acc)

        plsc.subcore_barrier()

        tab_r = tab.at[c] if col_split else tab.at[0]
        if col_split:
            start = s
            step = _NS
        else:
            start = c * _NS + s
            step = 2 * _NS
        n_my = (_NSUP - start + step - 1) // step

        def body(i, carry):
            u = start + i * step
            pltpu.sync_copy(idx.at[:, pl.ds(u * _GRP, _GRP)], idx_v)
            handles = [
                pltpu.async_copy(tab_r.at[idx_v.at[0, b]], rows_v.at[b], sem)
                for b in range(_GRP)
            ]
            for h in handles:
                h.wait()
            for b in range(_GRP):
                pltpu.sync_copy(rows_v.at[b], acc.at[idx_v.at[1, b]],
                                add=True)
            return carry

        lax.fori_loop(0, n_my, body, 0)
        plsc.subcore_barrier()

        @pl.when(s == 0)
        def _write():
            pltpu.sync_copy(acc, out.at[c])

    return k


_seg_sum8 = _make_seg_sum(8, col_split=False)
_seg_sum32 = _make_seg_sum(32, col_split=True)


def _make_gat_edge():
    mesh = plsc.VectorSubcoreMesh(core_axis_name="c", subcore_axis_name="s")

    @functools.partial(
        pl.kernel,
        out_type=(jax.ShapeDtypeStruct((_NCH, _CH, 16), jnp.float32),
                  jax.ShapeDtypeStruct((2, N, 16), jnp.float32)),
        mesh=mesh,
        scratch_types=[
            pltpu.VMEM((2, _CH), jnp.int32),
            pltpu.VMEM((_CH, 16), jnp.float32),
            pltpu.VMEM((_CH, 16), jnp.float32),
            pltpu.VMEM((16,), jnp.float32),
            pltpu.VMEM_SHARED((N, 16), jnp.float32),
            pltpu.SemaphoreType.DMA,
        ],
        compiler_params=pltpu.CompilerParams(use_tc_tiling_on_sc=False),
    )
    def k(asrc, adst, mvec, idx, zeros, w_out, s_out, idx_v, a_v, b_v, m_v,
          acc, sem):
        c = lax.axis_index("c")
        s = lax.axis_index("s")

        @pl.when(s == 0)
        def _zero():
            pltpu.sync_copy(zeros, acc)

        pltpu.sync_copy(mvec, m_v)
        plsc.subcore_barrier()

        start = c * _NS + s
        step = 2 * _NS
        n_my = (_NCH - start + step - 1) // step

        def body(i, carry):
            j = start + i * step
            pltpu.sync_copy(idx.at[j], idx_v)
            pltpu.async_copy(asrc.at[idx_v.at[0]], a_v, sem).wait()
            pltpu.async_copy(adst.at[idx_v.at[1]], b_v, sem).wait()
            mv = m_v[...]

            def ebody(i2, c2):
                e = a_v[i2] + b_v[i2]
                e = jnp.maximum(e, 0.2 * e)
                a_v[i2] = jnp.exp(e - mv)
                return c2

            lax.fori_loop(0, _CH, ebody, 0)
            pltpu.sync_copy(a_v, acc.at[idx_v.at[1]], add=True)
            pltpu.sync_copy(a_v, w_out.at[j])
            return carry

        lax.fori_loop(0, n_my, body, 0)
        plsc.subcore_barrier()

        @pl.when(s == 0)
        def _write():
            pltpu.sync_copy(acc, s_out.at[c])

    return k


def _make_wseg(two_heads):
    mesh = plsc.VectorSubcoreMesh(core_axis_name="c", subcore_axis_name="s")

    @functools.partial(
        pl.kernel,
        out_type=jax.ShapeDtypeStruct((2, N, 32), jnp.float32),
        mesh=mesh,
        scratch_types=[
            pltpu.VMEM((2, _CH), jnp.int32),
            pltpu.VMEM((_CH, 32), jnp.float32),
            pltpu.VMEM((_CH, 16), jnp.float32),
            pltpu.VMEM_SHARED((N, 32), jnp.float32),
            pltpu.SemaphoreType.DMA,
        ],
        compiler_params=pltpu.CompilerParams(use_tc_tiling_on_sc=False),
    )
    def k(tab, w_in, idx, zeros, out, idx_v, rows_v, w_v, acc, sem):
        c = lax.axis_index("c")
        s = lax.axis_index("s")

        @pl.when(s == 0)
        def _zero():
            pltpu.sync_copy(zeros, acc)

        plsc.subcore_barrier()

        def body(i, carry):
            j = s + i * _NS
            pltpu.sync_copy(idx.at[j], idx_v)
            pltpu.async_copy(tab.at[c].at[idx_v.at[0]], rows_v, sem).wait()
            pltpu.sync_copy(w_in.at[j], w_v)

            def ebody(i2, c2):
                wrow = w_v[i2]
                if two_heads:
                    w0 = jnp.where(c == 0, wrow[0], wrow[2])
                    w1 = jnp.where(c == 0, wrow[1], wrow[3])
                else:
                    w0 = wrow[0]
                    w1 = w0
                rows_v[i2, pl.ds(0, 16)] = rows_v[i2, pl.ds(0, 16)] * w0
                rows_v[i2, pl.ds(16, 16)] = rows_v[i2, pl.ds(16, 16)] * w1
                return c2

            lax.fori_loop(0, _CH, ebody, 0)
            pltpu.sync_copy(rows_v, acc.at[idx_v.at[1]], add=True)
            return carry

        n_my = (_NCH - s + _NS - 1) // _NS
        lax.fori_loop(0, n_my, body, 0)
        plsc.subcore_barrier()

        @pl.when(s == 0)
        def _write():
            pltpu.sync_copy(acc, out.at[c])

    return k


_gat_edge = _make_gat_edge()
_wseg2 = _make_wseg(True)
_wseg1 = _make_wseg(False)


def _edge_chunks(ei):
    ei = ei.astype(jnp.int32)
    return jnp.transpose(ei.reshape(2, _NCH, _CH), (1, 0, 2))


def _edge_chunks_grp(ei):
    return ei.astype(jnp.int32).reshape(2, _NCH, _CH)


def _ln(x, g, b):
    mu = jnp.mean(x, axis=-1, keepdims=True)
    var = jnp.var(x, axis=-1, keepdims=True)
    return (x - mu) / jnp.sqrt(var + 1e-5) * g + b


_PB = 1000
_NPB = N // _PB


def _pool_fusion_body(xsp_ref, xkp_ref, spb_ref, kpb_ref, w1_ref, b1_ref,
                      g1_ref, be1_ref, w2_ref, b2_ref, cw1_ref, cb1_ref,
                      cg_ref, cbe_ref, cw2_ref, cb2_ref, out_ref,
                      accs, acck, cnts, cntk):
    i = pl.program_id(0)

    @pl.when(i == 0)
    def _zero():
        accs[...] = jnp.zeros_like(accs)
        acck[...] = jnp.zeros_like(acck)
        cnts[...] = jnp.zeros_like(cnts)
        cntk[...] = jnp.zeros_like(cntk)

    gids = lax.broadcasted_iota(jnp.int32, (G, _PB), 0)
    dims = (((1,), (0,)), ((), ()))
    ohs = (spb_ref[0] == gids).astype(jnp.float32)
    accs[...] += lax.dot_general(ohs, xsp_ref[...], dims,
                                 preferred_element_type=jnp.float32)
    cnts[...] += jnp.sum(ohs, axis=1, keepdims=True)
    ohk = (kpb_ref[0] == gids).astype(jnp.float32)
    acck[...] += lax.dot_general(ohk, xkp_ref[...], dims,
                                 preferred_element_type=jnp.float32)
    cntk[...] += jnp.sum(ohk, axis=1, keepdims=True)

    @pl.when(i == _NPB - 1)
    def _finish():
        out_sp = accs[...] / jnp.maximum(cnts[...], 1.0)
        out_kp = acck[...] / jnp.maximum(cntk[...], 1.0)

        def l2n(v):
            nrm = jnp.sqrt(jnp.sum(v * v, axis=1, keepdims=True))
            return v / jnp.maximum(nrm, 1e-12)

        out_sp = l2n(out_sp)
        out_kp = l2n(out_kp)
        gin = jnp.concatenate([out_sp, out_kp], axis=-1)
        g1 = gin @ w1_ref[...] + b1_ref[...]
        g1 = jax.nn.relu(_ln(g1, g1_ref[...], be1_ref[...]))
        gate = jax.nn.sigmoid(g1 @ w2_ref[...] + b2_ref[...])
        fused = gate * out_sp + (1.0 - gate) * out_kp
        h1 = fused @ cw1_ref[...] + cb1_ref[...]
        h1 = jax.nn.relu(_ln(h1, cg_ref[...], cbe_ref[...]))
        out_ref[...] = h1 @ cw2_ref[...] + cb2_ref[...]


def _pool_fusion(x_sp, sp_batch, x_kp, kp_batch, gp, cp):
    spb = sp_batch.astype(jnp.int32).reshape(_NPB, 1, _PB)
    kpb = kp_batch.astype(jnp.int32).reshape(_NPB, 1, _PB)
    args = [x_sp, x_kp, spb, kpb,
            gp['W1'], gp['b1'].reshape(1, H), gp['g'].reshape(1, H),
            gp['be'].reshape(1, H), gp['W2'], gp['b2'].reshape(1, H),
            cp['W1'], cp['b1'].reshape(1, H), cp['g'].reshape(1, H),
            cp['be'].reshape(1, H), cp['W2'], cp['b2'].reshape(1, C)]
    node_spec = pl.BlockSpec((_PB, H), lambda i: (i, 0))
    batch_spec = pl.BlockSpec((1, 1, _PB), lambda i: (i, 0, 0))

    def full(a):
        nd = a.ndim
        return pl.BlockSpec(a.shape, lambda i, _n=nd: (0,) * _n)

    in_specs = [node_spec, node_spec, batch_spec, batch_spec] + [
        full(a) for a in args[4:]]
    return pl.pallas_call(
        _pool_fusion_body,
        grid=(_NPB,),
        in_specs=in_specs,
        out_specs=pl.BlockSpec((G, C), lambda i: (0, 0)),
        out_shape=jax.ShapeDtypeStruct((G, C), jnp.float32),
        scratch_shapes=[
            pltpu.VMEM((G, H), jnp.float32),
            pltpu.VMEM((G, H), jnp.float32),
            pltpu.VMEM((G, 1), jnp.float32),
            pltpu.VMEM((G, 1), jnp.float32),
        ],
    )(*args)


def kernel(sp_x, sp_edge_index, sp_batch, kp_x, kp_edge_index, kp_batch,
           params):
    sp_chunks = _edge_chunks_grp(sp_edge_index)
    x = sp_x
    zeros8 = jnp.zeros((N, 8), jnp.float32)
    zeros32 = jnp.zeros((N, 32), jnp.float32)
    for li, lp in enumerate(params['sp']):
        if li == 0:
            tab = jnp.concatenate(
                [x, jnp.zeros((N, 1), jnp.float32),
                 jnp.ones((N, 1), jnp.float32)], axis=1)
            part = _seg_sum8(tab[None], sp_chunks, zeros8)
            agg = part[0] + part[1]
            s = agg[:, :6]
            cnt = agg[:, 7]
        else:
            tab = jnp.transpose(x.reshape(N, 2, 32), (1, 0, 2))
            halves = _seg_sum32(tab, sp_chunks, zeros32)
            s = jnp.concatenate([halves[0], halves[1]], axis=1)
        mean = s / jnp.maximum(cnt, 1.0)[:, None]
        x = mean @ lp['Wl'] + x @ lp['Wr'] + lp['b']
        x = jax.nn.relu(_ln(x, lp['g'], lp['be']))
    x_sp = x

    kp_chunks = _edge_chunks(kp_edge_index)
    x = kp_x
    zeros16 = jnp.zeros((N, 16), jnp.float32)
    kp_cfg = [(4, 16, True), (4, 16, True), (1, H, False)]
    for lp, (h, oh, cc) in zip(params['kp'], kp_cfg):
        hx = x @ lp['W']
        hxr = hx.reshape(N, h, oh)
        a_s = jnp.sum(hxr * lp['asrc'][None], axis=-1)
        a_d = jnp.sum(hxr * lp['adst'][None], axis=-1)
        m_h = jax.nn.leaky_relu(
            jnp.max(a_s, axis=0) + jnp.max(a_d, axis=0), 0.2)
        mvec = jnp.zeros((16,), jnp.float32).at[:h].set(m_h)
        pad = jnp.zeros((N, 16 - h), jnp.float32)
        asrc_tab = jnp.concatenate([a_s, pad], axis=1)
        adst_tab = jnp.concatenate([a_d, pad], axis=1)
        w_chunks, s_part = _gat_edge(asrc_tab, adst_tab, mvec, kp_chunks,
                                     zeros16)
        denom = (s_part[0] + s_part[1])[:, :h]
        tab = jnp.transpose(hx.reshape(N, 2, 32), (1, 0, 2))
        wseg = _wseg2 if h == 4 else _wseg1
        halves = wseg(tab, w_chunks, kp_chunks, zeros32)
        acc = jnp.concatenate([halves[0], halves[1]], axis=1)
        out = acc.reshape(N, h, oh) / (denom[:, :, None] + 1e-16)
        x = out.reshape(N, h * oh) if cc else out.mean(axis=1)
        x = x + lp['b']
        x = jax.nn.relu(_ln(x, lp['g'], lp['be']))

    return _pool_fusion(x_sp, sp_batch, x, kp_batch, params['gate'],
                        params['cls'])


# speedup vs baseline: 32.6236x; 1.2591x over previous
import functools

import jax
import jax.numpy as jnp
from jax import lax
from jax.experimental import pallas as pl
from jax.experimental.pallas import tpu as pltpu
from jax.experimental.pallas import tpu_sc as plsc

N = 50000
E = 800000
G = 64
H = 64
C = 100

_CH = 128
_NCH = E // _CH
_NS = 16


_GRP = 5
_NSUP = _NCH // _GRP


def _make_seg_sum(d, col_split):
    mesh = plsc.VectorSubcoreMesh(core_axis_name="c", subcore_axis_name="s")

    @functools.partial(
        pl.kernel,
        out_type=jax.ShapeDtypeStruct((2, N, d), jnp.float32),
        mesh=mesh,
        scratch_types=[
            pltpu.VMEM((2, _GRP, _CH), jnp.int32),
            pltpu.VMEM((_GRP, _CH, d), jnp.float32),
            pltpu.VMEM_SHARED((N, d), jnp.float32),
            pltpu.SemaphoreType.DMA,
        ],
        compiler_params=pltpu.CompilerParams(use_tc_tiling_on_sc=False),
    )
    def k(tab, idx, zeros, out, idx_v, rows_v, acc, sem):
        c = lax.axis_index("c")
        s = lax.axis_index("s")

        @pl.when(s == 0)
        def _zero():
            pltpu.sync_copy(zeros, acc)

        plsc.subcore_barrier()

        tab_r = tab.at[c] if col_split else tab.at[0]
        if col_split:
            start = s
            step = _NS
        else:
            start = c * _NS + s
            step = 2 * _NS
        n_my = (_NSUP - start + step - 1) // step

        def body(i, carry):
            u = start + i * step
            pltpu.sync_copy(idx.at[:, pl.ds(u * _GRP, _GRP)], idx_v)
            handles = [
                pltpu.async_copy(tab_r.at[idx_v.at[0, b]], rows_v.at[b], sem)
                for b in range(_GRP)
            ]
            for h in handles:
                h.wait()
            for b in range(_GRP):
                pltpu.sync_copy(rows_v.at[b], acc.at[idx_v.at[1, b]],
                                add=True)
            return carry

        lax.fori_loop(0, n_my, body, 0)
        plsc.subcore_barrier()

        @pl.when(s == 0)
        def _write():
            pltpu.sync_copy(acc, out.at[c])

    return k


_seg_sum8 = _make_seg_sum(8, col_split=False)
_seg_sum32 = _make_seg_sum(32, col_split=True)


def _make_gat_edge():
    mesh = plsc.VectorSubcoreMesh(core_axis_name="c", subcore_axis_name="s")

    @functools.partial(
        pl.kernel,
        out_type=(jax.ShapeDtypeStruct((_NCH, _CH, 16), jnp.float32),
                  jax.ShapeDtypeStruct((2, N, 16), jnp.float32)),
        mesh=mesh,
        scratch_types=[
            pltpu.VMEM((2, _GRP, _CH), jnp.int32),
            pltpu.VMEM((_GRP, _CH, 16), jnp.float32),
            pltpu.VMEM((_GRP, _CH, 16), jnp.float32),
            pltpu.VMEM((16,), jnp.float32),
            pltpu.VMEM_SHARED((N, 16), jnp.float32),
            pltpu.SemaphoreType.DMA,
        ],
        compiler_params=pltpu.CompilerParams(use_tc_tiling_on_sc=False),
    )
    def k(asrc, adst, mvec, idx, zeros, w_out, s_out, idx_v, a_v, b_v, m_v,
          acc, sem):
        c = lax.axis_index("c")
        s = lax.axis_index("s")

        @pl.when(s == 0)
        def _zero():
            pltpu.sync_copy(zeros, acc)

        pltpu.sync_copy(mvec, m_v)
        plsc.subcore_barrier()

        start = c * _NS + s
        step = 2 * _NS
        n_my = (_NSUP - start + step - 1) // step

        def body(i, carry):
            u = start + i * step
            pltpu.sync_copy(idx.at[:, pl.ds(u * _GRP, _GRP)], idx_v)
            handles = [
                pltpu.async_copy(asrc.at[idx_v.at[0, b]], a_v.at[b], sem)
                for b in range(_GRP)
            ] + [
                pltpu.async_copy(adst.at[idx_v.at[1, b]], b_v.at[b], sem)
                for b in range(_GRP)
            ]
            for h in handles:
                h.wait()
            mv = m_v[...]
            for b in range(_GRP):
                def ebody(i2, c2, _b=b):
                    e = a_v[_b, i2] + b_v[_b, i2]
                    e = jnp.maximum(e, 0.2 * e)
                    a_v[_b, i2] = jnp.exp(e - mv)
                    return c2

                lax.fori_loop(0, _CH, ebody, 0)
            for b in range(_GRP):
                pltpu.sync_copy(a_v.at[b], acc.at[idx_v.at[1, b]], add=True)
            pltpu.sync_copy(a_v, w_out.at[pl.ds(u * _GRP, _GRP)])
            return carry

        lax.fori_loop(0, n_my, body, 0)
        plsc.subcore_barrier()

        @pl.when(s == 0)
        def _write():
            pltpu.sync_copy(acc, s_out.at[c])

    return k


def _make_wseg(two_heads):
    mesh = plsc.VectorSubcoreMesh(core_axis_name="c", subcore_axis_name="s")

    @functools.partial(
        pl.kernel,
        out_type=jax.ShapeDtypeStruct((2, N, 32), jnp.float32),
        mesh=mesh,
        scratch_types=[
            pltpu.VMEM((2, _GRP, _CH), jnp.int32),
            pltpu.VMEM((_GRP, _CH, 32), jnp.float32),
            pltpu.VMEM((_CH, 16), jnp.float32),
            pltpu.VMEM_SHARED((N, 32), jnp.float32),
            pltpu.SemaphoreType.DMA,
        ],
        compiler_params=pltpu.CompilerParams(use_tc_tiling_on_sc=False),
    )
    def k(tab, w_in, idx, zeros, out, idx_v, rows_v, w_v, acc, sem):
        c = lax.axis_index("c")
        s = lax.axis_index("s")

        @pl.when(s == 0)
        def _zero():
            pltpu.sync_copy(zeros, acc)

        plsc.subcore_barrier()

        def body(i, carry):
            u = s + i * _NS
            pltpu.sync_copy(idx.at[:, pl.ds(u * _GRP, _GRP)], idx_v)
            handles = [
                pltpu.async_copy(tab.at[c].at[idx_v.at[0, b]], rows_v.at[b],
                                 sem)
                for b in range(_GRP)
            ]
            for h in handles:
                h.wait()
            for b in range(_GRP):
                pltpu.sync_copy(w_in.at[u * _GRP + b], w_v)

                def ebody(i2, c2, _b=b):
                    wrow = w_v[i2]
                    if two_heads:
                        w0 = jnp.where(c == 0, wrow[0], wrow[2])
                        w1 = jnp.where(c == 0, wrow[1], wrow[3])
                    else:
                        w0 = wrow[0]
                        w1 = w0
                    rows_v[_b, i2, pl.ds(0, 16)] = (
                        rows_v[_b, i2, pl.ds(0, 16)] * w0)
                    rows_v[_b, i2, pl.ds(16, 16)] = (
                        rows_v[_b, i2, pl.ds(16, 16)] * w1)
                    return c2

                lax.fori_loop(0, _CH, ebody, 0)
            for b in range(_GRP):
                pltpu.sync_copy(rows_v.at[b], acc.at[idx_v.at[1, b]],
                                add=True)
            return carry

        n_my = (_NSUP - s + _NS - 1) // _NS
        lax.fori_loop(0, n_my, body, 0)
        plsc.subcore_barrier()

        @pl.when(s == 0)
        def _write():
            pltpu.sync_copy(acc, out.at[c])

    return k


_gat_edge = _make_gat_edge()
_wseg2 = _make_wseg(True)
_wseg1 = _make_wseg(False)


def _edge_chunks_grp(ei):
    return ei.astype(jnp.int32).reshape(2, _NCH, _CH)


def _ln(x, g, b):
    mu = jnp.mean(x, axis=-1, keepdims=True)
    var = jnp.var(x, axis=-1, keepdims=True)
    return (x - mu) / jnp.sqrt(var + 1e-5) * g + b


_PB = 1000
_NPB = N // _PB


def _pool_fusion_body(xsp_ref, xkp_ref, spb_ref, kpb_ref, w1_ref, b1_ref,
                      g1_ref, be1_ref, w2_ref, b2_ref, cw1_ref, cb1_ref,
                      cg_ref, cbe_ref, cw2_ref, cb2_ref, out_ref,
                      accs, acck, cnts, cntk):
    i = pl.program_id(0)

    @pl.when(i == 0)
    def _zero():
        accs[...] = jnp.zeros_like(accs)
        acck[...] = jnp.zeros_like(acck)
        cnts[...] = jnp.zeros_like(cnts)
        cntk[...] = jnp.zeros_like(cntk)

    gids = lax.broadcasted_iota(jnp.int32, (G, _PB), 0)
    dims = (((1,), (0,)), ((), ()))
    ohs = (spb_ref[0] == gids).astype(jnp.float32)
    accs[...] += lax.dot_general(ohs, xsp_ref[...], dims,
                                 preferred_element_type=jnp.float32)
    cnts[...] += jnp.sum(ohs, axis=1, keepdims=True)
    ohk = (kpb_ref[0] == gids).astype(jnp.float32)
    acck[...] += lax.dot_general(ohk, xkp_ref[...], dims,
                                 preferred_element_type=jnp.float32)
    cntk[...] += jnp.sum(ohk, axis=1, keepdims=True)

    @pl.when(i == _NPB - 1)
    def _finish():
        out_sp = accs[...] / jnp.maximum(cnts[...], 1.0)
        out_kp = acck[...] / jnp.maximum(cntk[...], 1.0)

        def l2n(v):
            nrm = jnp.sqrt(jnp.sum(v * v, axis=1, keepdims=True))
            return v / jnp.maximum(nrm, 1e-12)

        out_sp = l2n(out_sp)
        out_kp = l2n(out_kp)
        gin = jnp.concatenate([out_sp, out_kp], axis=-1)
        g1 = gin @ w1_ref[...] + b1_ref[...]
        g1 = jax.nn.relu(_ln(g1, g1_ref[...], be1_ref[...]))
        gate = jax.nn.sigmoid(g1 @ w2_ref[...] + b2_ref[...])
        fused = gate * out_sp + (1.0 - gate) * out_kp
        h1 = fused @ cw1_ref[...] + cb1_ref[...]
        h1 = jax.nn.relu(_ln(h1, cg_ref[...], cbe_ref[...]))
        out_ref[...] = h1 @ cw2_ref[...] + cb2_ref[...]


def _pool_fusion(x_sp, sp_batch, x_kp, kp_batch, gp, cp):
    spb = sp_batch.astype(jnp.int32).reshape(_NPB, 1, _PB)
    kpb = kp_batch.astype(jnp.int32).reshape(_NPB, 1, _PB)
    args = [x_sp, x_kp, spb, kpb,
            gp['W1'], gp['b1'].reshape(1, H), gp['g'].reshape(1, H),
            gp['be'].reshape(1, H), gp['W2'], gp['b2'].reshape(1, H),
            cp['W1'], cp['b1'].reshape(1, H), cp['g'].reshape(1, H),
            cp['be'].reshape(1, H), cp['W2'], cp['b2'].reshape(1, C)]
    node_spec = pl.BlockSpec((_PB, H), lambda i: (i, 0))
    batch_spec = pl.BlockSpec((1, 1, _PB), lambda i: (i, 0, 0))

    def full(a):
        nd = a.ndim
        return pl.BlockSpec(a.shape, lambda i, _n=nd: (0,) * _n)

    in_specs = [node_spec, node_spec, batch_spec, batch_spec] + [
        full(a) for a in args[4:]]
    return pl.pallas_call(
        _pool_fusion_body,
        grid=(_NPB,),
        in_specs=in_specs,
        out_specs=pl.BlockSpec((G, C), lambda i: (0, 0)),
        out_shape=jax.ShapeDtypeStruct((G, C), jnp.float32),
        scratch_shapes=[
            pltpu.VMEM((G, H), jnp.float32),
            pltpu.VMEM((G, H), jnp.float32),
            pltpu.VMEM((G, 1), jnp.float32),
            pltpu.VMEM((G, 1), jnp.float32),
        ],
    )(*args)


def kernel(sp_x, sp_edge_index, sp_batch, kp_x, kp_edge_index, kp_batch,
           params):
    sp_chunks = _edge_chunks_grp(sp_edge_index)
    x = sp_x
    zeros8 = jnp.zeros((N, 8), jnp.float32)
    zeros32 = jnp.zeros((N, 32), jnp.float32)
    for li, lp in enumerate(params['sp']):
        if li == 0:
            tab = jnp.concatenate(
                [x, jnp.zeros((N, 1), jnp.float32),
                 jnp.ones((N, 1), jnp.float32)], axis=1)
            part = _seg_sum8(tab[None], sp_chunks, zeros8)
            agg = part[0] + part[1]
            s = agg[:, :6]
            cnt = agg[:, 7]
        else:
            tab = jnp.transpose(x.reshape(N, 2, 32), (1, 0, 2))
            halves = _seg_sum32(tab, sp_chunks, zeros32)
            s = jnp.concatenate([halves[0], halves[1]], axis=1)
        mean = s / jnp.maximum(cnt, 1.0)[:, None]
        x = mean @ lp['Wl'] + x @ lp['Wr'] + lp['b']
        x = jax.nn.relu(_ln(x, lp['g'], lp['be']))
    x_sp = x

    kp_chunks = _edge_chunks_grp(kp_edge_index)
    x = kp_x
    zeros16 = jnp.zeros((N, 16), jnp.float32)
    kp_cfg = [(4, 16, True), (4, 16, True), (1, H, False)]
    for lp, (h, oh, cc) in zip(params['kp'], kp_cfg):
        hx = x @ lp['W']
        hxr = hx.reshape(N, h, oh)
        a_s = jnp.sum(hxr * lp['asrc'][None], axis=-1)
        a_d = jnp.sum(hxr * lp['adst'][None], axis=-1)
        m_h = jax.nn.leaky_relu(
            jnp.max(a_s, axis=0) + jnp.max(a_d, axis=0), 0.2)
        mvec = jnp.zeros((16,), jnp.float32).at[:h].set(m_h)
        pad = jnp.zeros((N, 16 - h), jnp.float32)
        asrc_tab = jnp.concatenate([a_s, pad], axis=1)
        adst_tab = jnp.concatenate([a_d, pad], axis=1)
        w_chunks, s_part = _gat_edge(asrc_tab, adst_tab, mvec, kp_chunks,
                                     zeros16)
        denom = (s_part[0] + s_part[1])[:, :h]
        tab = jnp.transpose(hx.reshape(N, 2, 32), (1, 0, 2))
        wseg = _wseg2 if h == 4 else _wseg1
        halves = wseg(tab, w_chunks, kp_chunks, zeros32)
        acc = jnp.concatenate([halves[0], halves[1]], axis=1)
        out = acc.reshape(N, h, oh) / (denom[:, :, None] + 1e-16)
        x = out.reshape(N, h * oh) if cc else out.mean(axis=1)
        x = x + lp['b']
        x = jax.nn.relu(_ln(x, lp['g'], lp['be']))

    return _pool_fusion(x_sp, sp_batch, x, kp_batch, params['gate'],
                        params['cls'])
